# baseline (speedup 1.0000x reference)
import functools

import jax
import jax.numpy as jnp
from jax import lax
from jax.experimental import pallas as pl
from jax.experimental.pallas import tpu as pltpu
from jax.experimental.pallas import tpu_sc as plsc

NU = 5000
NI = 5000
N = NU + NI
D = 256
NLAYERS = 3
E = 150000
B = 4096
NC = 2
NS = 16
NW = NC * NS
CHUNK = 64
NSPH = 5120
NSP2 = 2 * NSPH
ZPH = NSP2 // NS
OUT_FULL = 640
OUT_TOFF = 15 * OUT_FULL
OUT_TAIL = 2 * NU - OUT_TOFF
CPW = 80
EHP = NW * CPW * CHUNK
NPI = 8
ROUNDS = CPW // NPI
PPW = B // NW
RBLK = 1000
NBLK = N // RBLK

_f32 = jnp.float32


@functools.cache
def _mesh():
    return plsc.VectorSubcoreMesh(core_axis_name="c", subcore_axis_name="s")


def _make_agg(with_deg):
    out_type = [jax.ShapeDtypeStruct((NC * 2 * N, 128), _f32)]
    scratch = [
        pltpu.VMEM((NPI, 128), jnp.int32),
        pltpu.VMEM((128,), jnp.int32),
        pltpu.VMEM((128,), jnp.int32),
        pltpu.VMEM((128, 128), _f32),
        pltpu.VMEM((128, 128), _f32),
        pltpu.VMEM_SHARED((NSP2, 128), _f32),
        pltpu.SemaphoreType.DMA,
        pltpu.SemaphoreType.DMA,
        pltpu.SemaphoreType.DMA,
    ]
    if with_deg:
        out_type.append(jax.ShapeDtypeStruct((NC * 2 * N,), _f32))
        scratch += [
            pltpu.VMEM((128,), _f32),
            pltpu.VMEM_SHARED((NSP2,), _f32),
            pltpu.VMEM((ZPH,), _f32),
        ]

    def body(xw, srcall, dstall, z2, z1, *refs):
        if with_deg:
            (aggout, degout, sidxa, didx0, didx1, b0, b1, shared,
             gs0, gs1, ssem, ones, degsh, degv) = refs
        else:
            (aggout, sidxa, didx0, didx1, b0, b1, shared, gs0, gs1, ssem) = refs
        c = lax.axis_index("c")
        s = lax.axis_index("s")
        w = c * NS + s
        b0w = b0.reshape(CHUNK, D)
        b1w = b1.reshape(CHUNK, D)
        if with_deg:
            for k in range(128 // 16):
                ones[pl.ds(k * 16, 16)] = jnp.ones((16,), _f32)

        for p in range(2):
            out_base = c * 2 * N + (2 * NU if p == 0 else 0)
            pltpu.sync_copy(z2.at[pl.ds(s * ZPH, ZPH)], shared.at[pl.ds(s * ZPH, ZPH)])
            if with_deg:
                pltpu.sync_copy(z1.at[pl.ds(s * ZPH, ZPH)], degv)
                pltpu.sync_copy(degv, degsh.at[pl.ds(s * ZPH, ZPH)])
            plsc.subcore_barrier()

            def iter_(t, carry):
                row0 = p * (NW * CPW) + w * CPW + t * NPI
                pltpu.sync_copy(srcall.at[pl.ds(row0, NPI)], sidxa)
                for q in range(NPI // 2):
                    j0, j1 = 2 * q, 2 * q + 1
                    pltpu.async_copy(
                        xw.at[sidxa.at[j0, pl.ds(0, CHUNK)]], b0w, gs0)
                    pltpu.async_copy(
                        xw.at[sidxa.at[j1, pl.ds(0, CHUNK)]], b1w, gs1)
                    pltpu.sync_copy(dstall.at[row0 + j0], didx0)
                    pltpu.sync_copy(dstall.at[row0 + j1], didx1)
                    pltpu.make_async_copy(
                        xw.at[pl.ds(0, CHUNK)], b0w, gs0).wait()
                    pltpu.async_copy(b0, shared.at[didx0], ssem, add=True)
                    if with_deg:
                        pltpu.async_copy(ones, degsh.at[didx0], ssem, add=True)
                    pltpu.make_async_copy(
                        xw.at[pl.ds(0, CHUNK)], b1w, gs1).wait()
                    pltpu.async_copy(b1, shared.at[didx1], ssem, add=True)
                    if with_deg:
                        pltpu.async_copy(ones, degsh.at[didx1], ssem, add=True)
                    pltpu.make_async_copy(b0, shared.at[didx0], ssem).wait()
                    pltpu.make_async_copy(b1, shared.at[didx1], ssem).wait()
                    if with_deg:
                        pltpu.make_async_copy(ones, degsh.at[didx0], ssem).wait()
                        pltpu.make_async_copy(ones, degsh.at[didx1], ssem).wait()
                return carry

            lax.fori_loop(0, ROUNDS, iter_, 0)
            plsc.subcore_barrier()

            @pl.when(s < NS - 1)
            def _():
                pltpu.sync_copy(shared.at[pl.ds(s * OUT_FULL, OUT_FULL)],
                                aggout.at[pl.ds(out_base + s * OUT_FULL, OUT_FULL)])

            @pl.when(s == NS - 1)
            def _():
                pltpu.sync_copy(shared.at[pl.ds(OUT_TOFF, OUT_TAIL)],
                                aggout.at[pl.ds(out_base + OUT_TOFF, OUT_TAIL)])

            if with_deg:
                @pl.when(s < NS - 1)
                def _():
                    pltpu.sync_copy(degsh.at[pl.ds(s * OUT_FULL, OUT_FULL)], degv.at[pl.ds(0, OUT_FULL)])
                    pltpu.sync_copy(degv.at[pl.ds(0, OUT_FULL)],
                                    degout.at[pl.ds(out_base + s * OUT_FULL, OUT_FULL)])

                @pl.when(s == NS - 1)
                def _():
                    pltpu.sync_copy(degsh.at[pl.ds(OUT_TOFF, OUT_TAIL)], degv.at[pl.ds(0, OUT_TAIL)])
                    pltpu.sync_copy(degv.at[pl.ds(0, OUT_TAIL)],
                                    degout.at[pl.ds(out_base + OUT_TOFF, OUT_TAIL)])

            plsc.subcore_barrier()

    return pl.kernel(body, mesh=_mesh(), out_type=tuple(out_type) if with_deg else out_type[0],
                     scratch_types=scratch)


@functools.cache
def _agg_deg():
    return _make_agg(True)


@functools.cache
def _agg():
    return _make_agg(False)


def _score_body(xw, u0, i0, out, uv, iv, xu, xi, sv, sem):
    c = lax.axis_index("c")
    s = lax.axis_index("s")
    base = (s * NC + c) * PPW
    pltpu.sync_copy(u0.at[pl.ds(base, PPW)], uv)
    pltpu.sync_copy(i0.at[pl.ds(base, PPW)], iv)
    pltpu.async_copy(xw.at[uv], xu, sem).wait()
    pltpu.async_copy(xw.at[iv], xi, sem).wait()

    def pair(p, carry):
        acc = jnp.zeros((16,), _f32)
        for k in range(D // 16):
            sl = pl.ds(k * 16, 16)
            acc = acc + xu[p, sl] * xi[p, sl]
        sv[p] = acc
        return carry

    lax.fori_loop(0, PPW, pair, 0)
    pltpu.sync_copy(sv, out.at[pl.ds(base, PPW)])


@functools.cache
def _make_score():
    return pl.kernel(
        _score_body, mesh=_mesh(),
        out_type=jax.ShapeDtypeStruct((B, 16), _f32),
        scratch_types=[
            pltpu.VMEM((PPW,), jnp.int32),
            pltpu.VMEM((PPW,), jnp.int32),
            pltpu.VMEM((PPW, D), _f32),
            pltpu.VMEM((PPW, D), _f32),
            pltpu.VMEM((PPW, 16), _f32),
            pltpu.SemaphoreType.DMA,
        ])


def _finish_body(pp_ref, out_ref):
    s = jnp.sum(pp_ref[...], axis=1, keepdims=True)
    out_ref[...] = jnp.broadcast_to(1.0 / (1.0 + jnp.exp(-s)), (B, 128))


def _finish(pp):
    return pl.pallas_call(
        _finish_body,
        grid=(1,),
        in_specs=[pl.BlockSpec((B, 16), lambda b: (0, 0))],
        out_specs=pl.BlockSpec((B, 128), lambda b: (0, 0)),
        out_shape=jax.ShapeDtypeStruct((B, 128), _f32),
    )(pp)


def _enc_body(feat_ref, w_ref, b_ref, out_ref):
    out_ref[...] = (jnp.dot(feat_ref[...], w_ref[...], preferred_element_type=_f32)
                    + b_ref[0:1, :])


def _enc(item_feat, enc_W, encb_pk):
    return pl.pallas_call(
        _enc_body,
        grid=(NI // RBLK,),
        in_specs=[
            pl.BlockSpec((RBLK, D), lambda b: (b, 0)),
            pl.BlockSpec((D, D), lambda b: (0, 0)),
            pl.BlockSpec((8, D), lambda b: (0, 0)),
        ],
        out_specs=pl.BlockSpec((RBLK, D), lambda b: (b, 0)),
        out_shape=jax.ShapeDtypeStruct((NI, D), _f32),
    )(item_feat, enc_W, encb_pk)


def _dense_body(agg_ref, x_ref, deg_ref, wl_ref, wr_ref, pk_ref, y_ref, st_ref):
    b = pl.program_id(0)
    deg = deg_ref[0] + deg_ref[1]
    inv = 1.0 / jnp.maximum(deg, 1.0)
    a = (agg_ref[0] + agg_ref[1]) * inv
    y = (jnp.dot(a, wl_ref[...], preferred_element_type=_f32)
         + jnp.dot(x_ref[...], wr_ref[...], preferred_element_type=_f32)
         + pk_ref[0:1, :])
    y_ref[...] = y

    @pl.when(b == 0)
    def _():
        st_ref[...] = jnp.zeros_like(st_ref)

    st_ref[...] += jnp.concatenate(
        [jnp.sum(y, axis=0, keepdims=True),
         jnp.sum(y * y, axis=0, keepdims=True),
         jnp.zeros((6, D), _f32)], axis=0)


def _dense(aggp, x, degp, wl, wr, pk):
    return pl.pallas_call(
        _dense_body,
        grid=(NBLK,),
        in_specs=[
            pl.BlockSpec((NC, RBLK, D), lambda b: (0, b, 0)),
            pl.BlockSpec((RBLK, D), lambda b: (b, 0)),
            pl.BlockSpec((NC, RBLK, 1), lambda b: (0, b, 0)),
            pl.BlockSpec((D, D), lambda b: (0, 0)),
            pl.BlockSpec((D, D), lambda b: (0, 0)),
            pl.BlockSpec((8, D), lambda b: (0, 0)),
        ],
        out_specs=[
            pl.BlockSpec((RBLK, D), lambda b: (b, 0)),
            pl.BlockSpec((8, D), lambda b: (0, 0)),
        ],
        out_shape=[
            jax.ShapeDtypeStruct((N, D), _f32),
            jax.ShapeDtypeStruct((8, D), _f32),
        ],
    )(aggp, x, degp, wl, wr, pk)


def _make_norm(relu):
    def body(y_ref, st_ref, pk_ref, out_ref):
        mean = st_ref[0:1, :] * (1.0 / N)
        ey2 = st_ref[1:2, :] * (1.0 / N)
        var = ey2 - mean * mean
        rstd = lax.rsqrt(var + 1e-5)
        scale = pk_ref[1:2, :] * rstd
        shift = pk_ref[2:3, :] - mean * scale
        part = y_ref[...] * scale + shift
        if relu:
            part = jnp.maximum(part, 0.0)
        out_ref[...] = part

    def call(y, st, pk):
        return pl.pallas_call(
            body,
            grid=(NBLK,),
            in_specs=[
                pl.BlockSpec((RBLK, D), lambda b: (b, 0)),
                pl.BlockSpec((8, D), lambda b: (0, 0)),
                pl.BlockSpec((8, D), lambda b: (0, 0)),
            ],
            out_specs=pl.BlockSpec((RBLK, D), lambda b: (b, 0)),
            out_shape=jax.ShapeDtypeStruct((N, D), _f32),
        )(y, st, pk)

    return call


_norm_relu = _make_norm(True)
_norm_id = _make_norm(False)


def kernel(users, items, edge_user, edge_item, item_feat, user_emb,
           enc_W, enc_b, Wl, Wr, bl, gamma, beta):
    i32 = jnp.int32
    eu = edge_user.astype(i32)
    ei = edge_item.astype(i32)
    pad = EHP - E
    trash = NU + jnp.arange(pad, dtype=i32) % (NSPH - NU)
    zpad = jnp.zeros((pad,), i32)
    src64 = jnp.concatenate([eu, zpad, ei + NU, zpad]).reshape(-1, CHUNK)
    srcall = jnp.concatenate([src64, jnp.zeros_like(src64)], axis=1)
    dl = jnp.concatenate([ei, trash, eu, trash])
    dstall = jnp.stack([2 * dl, 2 * dl + 1], axis=1).reshape(-1, 128)

    z2 = jnp.zeros((NSP2, 128), _f32)
    z1 = jnp.zeros((NSP2,), _f32)

    zrow = jnp.zeros((1, D), _f32)
    pk = []
    for i in range(NLAYERS):
        pk.append(jnp.concatenate(
            [bl[i][None, :], gamma[i][None, :], beta[i][None, :],
             jnp.zeros((5, D), _f32)], axis=0))
    encb_pk = jnp.concatenate([enc_b[None, :]] + [zrow] * 7, axis=0)

    xi = _enc(item_feat, enc_W, encb_pk)
    x = jnp.concatenate([user_emb, xi], axis=0)

    degp = None
    for i in range(NLAYERS):
        if i == 0:
            aggflat, deg = _agg_deg()(x, srcall, dstall, z2, z1)
            degp = deg.reshape(NC, N, 2)[:, :, 0:1]
        else:
            aggflat = _agg()(x, srcall, dstall, z2, z1)
        aggp = aggflat.reshape(NC, N, D)
        y, st = _dense(aggp, x, degp, Wl[i], Wr[i], pk[i])
        x = (_norm_relu if i < NLAYERS - 1 else _norm_id)(y, st, pk[i])

    u0 = users.astype(i32)
    it0 = items.astype(i32) + NU
    pp = _make_score()(x, u0, it0)
    return _finish(pp)[:, 0]


# speedup vs baseline: 1.0013x; 1.0013x over previous
import functools

import jax
import jax.numpy as jnp
from jax import lax
from jax.experimental import pallas as pl
from jax.experimental.pallas import tpu as pltpu
from jax.experimental.pallas import tpu_sc as plsc

NU = 5000
NI = 5000
N = NU + NI
D = 256
NLAYERS = 3
E = 150000
B = 4096
NC = 2
NS = 16
NW = NC * NS
CHUNK = 64
NSPH = 5120
NSP2 = 2 * NSPH
ZPH = NSP2 // NS
OUT_FULL = 640
OUT_TOFF = 15 * OUT_FULL
OUT_TAIL = 2 * NU - OUT_TOFF
CPW = 80
EHP = NW * CPW * CHUNK
NPI = 8
ROUNDS = CPW // NPI
PPW = B // NW
RBLK = 1000
NBLK = N // RBLK

_f32 = jnp.float32


@functools.cache
def _mesh():
    return plsc.VectorSubcoreMesh(core_axis_name="c", subcore_axis_name="s")


def _make_agg(with_deg):
    out_type = [jax.ShapeDtypeStruct((NC * 2 * N, 128), _f32)]
    scratch = [
        pltpu.VMEM((NPI, 128), jnp.int32),
        pltpu.VMEM((NPI, 128), jnp.int32),
        pltpu.VMEM((128,), jnp.int32),
        pltpu.VMEM((128,), jnp.int32),
        pltpu.VMEM((128, 128), _f32),
        pltpu.VMEM((128, 128), _f32),
        pltpu.VMEM_SHARED((NSP2, 128), _f32),
        pltpu.SemaphoreType.DMA,
        pltpu.SemaphoreType.DMA,
        pltpu.SemaphoreType.DMA,
    ]
    if with_deg:
        out_type.append(jax.ShapeDtypeStruct((NC * 2 * N,), _f32))
        scratch += [
            pltpu.VMEM((128,), _f32),
            pltpu.VMEM_SHARED((NSP2,), _f32),
            pltpu.VMEM((ZPH,), _f32),
        ]

    def body(xw, srcall, dstall, z2, z1, *refs):
        if with_deg:
            (aggout, degout, sidxa, didxa, didx0, didx1, b0, b1, shared,
             gs0, gs1, ssem, ones, degsh, degv) = refs
        else:
            (aggout, sidxa, didxa, didx0, didx1, b0, b1, shared, gs0, gs1, ssem) = refs
        c = lax.axis_index("c")
        s = lax.axis_index("s")
        w = c * NS + s
        b0w = b0.reshape(CHUNK, D)
        b1w = b1.reshape(CHUNK, D)
        if with_deg:
            for k in range(128 // 16):
                ones[pl.ds(k * 16, 16)] = jnp.ones((16,), _f32)

        for p in range(2):
            out_base = c * 2 * N + (2 * NU if p == 0 else 0)
            pltpu.sync_copy(z2.at[pl.ds(s * ZPH, ZPH)], shared.at[pl.ds(s * ZPH, ZPH)])
            if with_deg:
                pltpu.sync_copy(z1.at[pl.ds(s * ZPH, ZPH)], degv)
                pltpu.sync_copy(degv, degsh.at[pl.ds(s * ZPH, ZPH)])
            plsc.subcore_barrier()

            def iter_(t, carry):
                row0 = p * (NW * CPW) + w * CPW + t * NPI
                pltpu.sync_copy(srcall.at[pl.ds(row0, NPI)], sidxa)
                pltpu.sync_copy(dstall.at[pl.ds(row0, NPI)], didxa)
                for q in range(NPI // 2):
                    j0, j1 = 2 * q, 2 * q + 1
                    pltpu.async_copy(
                        xw.at[sidxa.at[j0, pl.ds(0, CHUNK)]], b0w, gs0)
                    pltpu.async_copy(
                        xw.at[sidxa.at[j1, pl.ds(0, CHUNK)]], b1w, gs1)
                    for k in range(128 // 16):
                        sl = pl.ds(k * 16, 16)
                        didx0[sl] = didxa[j0, sl]
                        didx1[sl] = didxa[j1, sl]
                    pltpu.make_async_copy(
                        xw.at[pl.ds(0, CHUNK)], b0w, gs0).wait()
                    pltpu.async_copy(b0, shared.at[didx0], ssem, add=True)
                    if with_deg:
                        pltpu.async_copy(ones, degsh.at[didx0], ssem, add=True)
                    pltpu.make_async_copy(
                        xw.at[pl.ds(0, CHUNK)], b1w, gs1).wait()
                    pltpu.async_copy(b1, shared.at[didx1], ssem, add=True)
                    if with_deg:
                        pltpu.async_copy(ones, degsh.at[didx1], ssem, add=True)
                    pltpu.make_async_copy(b0, shared.at[didx0], ssem).wait()
                    pltpu.make_async_copy(b1, shared.at[didx1], ssem).wait()
                    if with_deg:
                        pltpu.make_async_copy(ones, degsh.at[didx0], ssem).wait()
                        pltpu.make_async_copy(ones, degsh.at[didx1], ssem).wait()
                return carry

            lax.fori_loop(0, ROUNDS, iter_, 0)
            plsc.subcore_barrier()

            @pl.when(s < NS - 1)
            def _():
                pltpu.sync_copy(shared.at[pl.ds(s * OUT_FULL, OUT_FULL)],
                                aggout.at[pl.ds(out_base + s * OUT_FULL, OUT_FULL)])

            @pl.when(s == NS - 1)
            def _():
                pltpu.sync_copy(shared.at[pl.ds(OUT_TOFF, OUT_TAIL)],
                                aggout.at[pl.ds(out_base + OUT_TOFF, OUT_TAIL)])

            if with_deg:
                @pl.when(s < NS - 1)
                def _():
                    pltpu.sync_copy(degsh.at[pl.ds(s * OUT_FULL, OUT_FULL)], degv.at[pl.ds(0, OUT_FULL)])
                    pltpu.sync_copy(degv.at[pl.ds(0, OUT_FULL)],
                                    degout.at[pl.ds(out_base + s * OUT_FULL, OUT_FULL)])

                @pl.when(s == NS - 1)
                def _():
                    pltpu.sync_copy(degsh.at[pl.ds(OUT_TOFF, OUT_TAIL)], degv.at[pl.ds(0, OUT_TAIL)])
                    pltpu.sync_copy(degv.at[pl.ds(0, OUT_TAIL)],
                                    degout.at[pl.ds(out_base + OUT_TOFF, OUT_TAIL)])

            plsc.subcore_barrier()

    return pl.kernel(body, mesh=_mesh(), out_type=tuple(out_type) if with_deg else out_type[0],
                     scratch_types=scratch)


@functools.cache
def _agg_deg():
    return _make_agg(True)


@functools.cache
def _agg():
    return _make_agg(False)


def _score_body(xw, u0, i0, out, uv, iv, xu, xi, sv, sem):
    c = lax.axis_index("c")
    s = lax.axis_index("s")
    base = (s * NC + c) * PPW
    pltpu.sync_copy(u0.at[pl.ds(base, PPW)], uv)
    pltpu.sync_copy(i0.at[pl.ds(base, PPW)], iv)
    pltpu.async_copy(xw.at[uv], xu, sem).wait()
    pltpu.async_copy(xw.at[iv], xi, sem).wait()

    def pair(p, carry):
        acc = jnp.zeros((16,), _f32)
        for k in range(D // 16):
            sl = pl.ds(k * 16, 16)
            acc = acc + xu[p, sl] * xi[p, sl]
        sv[p] = acc
        return carry

    lax.fori_loop(0, PPW, pair, 0)
    pltpu.sync_copy(sv, out.at[pl.ds(base, PPW)])


@functools.cache
def _make_score():
    return pl.kernel(
        _score_body, mesh=_mesh(),
        out_type=jax.ShapeDtypeStruct((B, 16), _f32),
        scratch_types=[
            pltpu.VMEM((PPW,), jnp.int32),
            pltpu.VMEM((PPW,), jnp.int32),
            pltpu.VMEM((PPW, D), _f32),
            pltpu.VMEM((PPW, D), _f32),
            pltpu.VMEM((PPW, 16), _f32),
            pltpu.SemaphoreType.DMA,
        ])


def _finish_body(pp_ref, out_ref):
    s = jnp.sum(pp_ref[...], axis=1, keepdims=True)
    out_ref[...] = jnp.broadcast_to(1.0 / (1.0 + jnp.exp(-s)), (B, 128))


def _finish(pp):
    return pl.pallas_call(
        _finish_body,
        grid=(1,),
        in_specs=[pl.BlockSpec((B, 16), lambda b: (0, 0))],
        out_specs=pl.BlockSpec((B, 128), lambda b: (0, 0)),
        out_shape=jax.ShapeDtypeStruct((B, 128), _f32),
    )(pp)


def _enc_body(feat_ref, w_ref, b_ref, out_ref):
    out_ref[...] = (jnp.dot(feat_ref[...], w_ref[...], preferred_element_type=_f32)
                    + b_ref[0:1, :])


def _enc(item_feat, enc_W, encb_pk):
    return pl.pallas_call(
        _enc_body,
        grid=(NI // RBLK,),
        in_specs=[
            pl.BlockSpec((RBLK, D), lambda b: (b, 0)),
            pl.BlockSpec((D, D), lambda b: (0, 0)),
            pl.BlockSpec((8, D), lambda b: (0, 0)),
        ],
        out_specs=pl.BlockSpec((RBLK, D), lambda b: (b, 0)),
        out_shape=jax.ShapeDtypeStruct((NI, D), _f32),
    )(item_feat, enc_W, encb_pk)


def _dense_body(agg_ref, x_ref, deg_ref, wl_ref, wr_ref, pk_ref, y_ref, st_ref):
    b = pl.program_id(0)
    deg = deg_ref[0] + deg_ref[1]
    inv = 1.0 / jnp.maximum(deg, 1.0)
    a = (agg_ref[0] + agg_ref[1]) * inv
    y = (jnp.dot(a, wl_ref[...], preferred_element_type=_f32)
         + jnp.dot(x_ref[...], wr_ref[...], preferred_element_type=_f32)
         + pk_ref[0:1, :])
    y_ref[...] = y

    @pl.when(b == 0)
    def _():
        st_ref[...] = jnp.zeros_like(st_ref)

    st_ref[...] += jnp.concatenate(
        [jnp.sum(y, axis=0, keepdims=True),
         jnp.sum(y * y, axis=0, keepdims=True),
         jnp.zeros((6, D), _f32)], axis=0)


def _dense(aggp, x, degp, wl, wr, pk):
    return pl.pallas_call(
        _dense_body,
        grid=(NBLK,),
        in_specs=[
            pl.BlockSpec((NC, RBLK, D), lambda b: (0, b, 0)),
            pl.BlockSpec((RBLK, D), lambda b: (b, 0)),
            pl.BlockSpec((NC, RBLK, 1), lambda b: (0, b, 0)),
            pl.BlockSpec((D, D), lambda b: (0, 0)),
            pl.BlockSpec((D, D), lambda b: (0, 0)),
            pl.BlockSpec((8, D), lambda b: (0, 0)),
        ],
        out_specs=[
            pl.BlockSpec((RBLK, D), lambda b: (b, 0)),
            pl.BlockSpec((8, D), lambda b: (0, 0)),
        ],
        out_shape=[
            jax.ShapeDtypeStruct((N, D), _f32),
            jax.ShapeDtypeStruct((8, D), _f32),
        ],
    )(aggp, x, degp, wl, wr, pk)


def _make_norm(relu):
    def body(y_ref, st_ref, pk_ref, out_ref):
        mean = st_ref[0:1, :] * (1.0 / N)
        ey2 = st_ref[1:2, :] * (1.0 / N)
        var = ey2 - mean * mean
        rstd = lax.rsqrt(var + 1e-5)
        scale = pk_ref[1:2, :] * rstd
        shift = pk_ref[2:3, :] - mean * scale
        part = y_ref[...] * scale + shift
        if relu:
            part = jnp.maximum(part, 0.0)
        out_ref[...] = part

    def call(y, st, pk):
        return pl.pallas_call(
            body,
            grid=(NBLK,),
            in_specs=[
                pl.BlockSpec((RBLK, D), lambda b: (b, 0)),
                pl.BlockSpec((8, D), lambda b: (0, 0)),
                pl.BlockSpec((8, D), lambda b: (0, 0)),
            ],
            out_specs=pl.BlockSpec((RBLK, D), lambda b: (b, 0)),
            out_shape=jax.ShapeDtypeStruct((N, D), _f32),
        )(y, st, pk)

    return call


_norm_relu = _make_norm(True)
_norm_id = _make_norm(False)


def kernel(users, items, edge_user, edge_item, item_feat, user_emb,
           enc_W, enc_b, Wl, Wr, bl, gamma, beta):
    i32 = jnp.int32
    eu = edge_user.astype(i32)
    ei = edge_item.astype(i32)
    pad = EHP - E
    trash = NU + jnp.arange(pad, dtype=i32) % (NSPH - NU)
    zpad = jnp.zeros((pad,), i32)
    src64 = jnp.concatenate([eu, zpad, ei + NU, zpad]).reshape(-1, CHUNK)
    srcall = jnp.concatenate([src64, jnp.zeros_like(src64)], axis=1)
    dl = jnp.concatenate([ei, trash, eu, trash])
    dstall = jnp.stack([2 * dl, 2 * dl + 1], axis=1).reshape(-1, 128)

    z2 = jnp.zeros((NSP2, 128), _f32)
    z1 = jnp.zeros((NSP2,), _f32)

    zrow = jnp.zeros((1, D), _f32)
    pk = []
    for i in range(NLAYERS):
        pk.append(jnp.concatenate(
            [bl[i][None, :], gamma[i][None, :], beta[i][None, :],
             jnp.zeros((5, D), _f32)], axis=0))
    encb_pk = jnp.concatenate([enc_b[None, :]] + [zrow] * 7, axis=0)

    xi = _enc(item_feat, enc_W, encb_pk)
    x = jnp.concatenate([user_emb, xi], axis=0)

    degp = None
    for i in range(NLAYERS):
        if i == 0:
            aggflat, deg = _agg_deg()(x, srcall, dstall, z2, z1)
            degp = deg.reshape(NC, N, 2)[:, :, 0:1]
        else:
            aggflat = _agg()(x, srcall, dstall, z2, z1)
        aggp = aggflat.reshape(NC, N, D)
        y, st = _dense(aggp, x, degp, Wl[i], Wr[i], pk[i])
        x = (_norm_relu if i < NLAYERS - 1 else _norm_id)(y, st, pk[i])

    u0 = users.astype(i32)
    it0 = items.astype(i32) + NU
    pp = _make_score()(x, u0, it0)
    return _finish(pp)[:, 0]


# speedup vs baseline: 3.5383x; 3.5337x over previous
import functools

import jax
import jax.numpy as jnp
from jax import lax
from jax.experimental import pallas as pl
from jax.experimental.pallas import tpu as pltpu
from jax.experimental.pallas import tpu_sc as plsc

NU = 5000
NI = 5000
N = NU + NI
D = 256
NLAYERS = 3
E = 150000
B = 4096
NC = 2
NS = 16
NW = NC * NS
CHUNK = 64
NSPH = 5120
NSP2 = 2 * NSPH
ZPH = NSP2 // NS
OUT_FULL = 640
OUT_TOFF = 15 * OUT_FULL
OUT_TAIL = 2 * NU - OUT_TOFF
CPW = 80
EHP = NW * CPW * CHUNK
NPI = 8
ROUNDS = CPW // NPI
PPW = B // NW
RBLK = 1000
NBLK = N // RBLK

_f32 = jnp.float32


@functools.cache
def _mesh():
    return plsc.VectorSubcoreMesh(core_axis_name="c", subcore_axis_name="s")


def _make_agg(with_deg):
    out_type = [jax.ShapeDtypeStruct((NC * 2 * N, 128), _f32)]
    scratch = [
        pltpu.VMEM((NPI, 128), jnp.int32),
        pltpu.VMEM((NPI, 128), jnp.int32),
        pltpu.VMEM((128,), jnp.int32),
        pltpu.VMEM((128,), jnp.int32),
        pltpu.VMEM((128, 128), _f32),
        pltpu.VMEM((128, 128), _f32),
        pltpu.VMEM_SHARED((NSP2, 128), _f32),
        pltpu.SemaphoreType.DMA,
        pltpu.SemaphoreType.DMA,
        pltpu.SemaphoreType.DMA,
    ]
    if with_deg:
        out_type.append(jax.ShapeDtypeStruct((NC * 2 * N,), _f32))
        scratch += [
            pltpu.VMEM((128,), _f32),
            pltpu.VMEM_SHARED((NSP2,), _f32),
            pltpu.VMEM((ZPH,), _f32),
        ]

    def body(xw, srcall, dstall, z2, z1, *refs):
        if with_deg:
            (aggout, degout, sidxa, didxa, didx0, didx1, b0, b1, shared,
             gs0, gs1, ssem, ones, degsh, degv) = refs
        else:
            (aggout, sidxa, didxa, didx0, didx1, b0, b1, shared, gs0, gs1, ssem) = refs
        c = lax.axis_index("c")
        s = lax.axis_index("s")
        w = c * NS + s
        b0w = b0.reshape(CHUNK, D)
        b1w = b1.reshape(CHUNK, D)
        if with_deg:
            for k in range(128 // 16):
                ones[pl.ds(k * 16, 16)] = jnp.ones((16,), _f32)

        for p in range(2):
            out_base = c * 2 * N + (2 * NU if p == 0 else 0)
            pltpu.sync_copy(z2.at[pl.ds(s * ZPH, ZPH)], shared.at[pl.ds(s * ZPH, ZPH)])
            if with_deg:
                pltpu.sync_copy(z1.at[pl.ds(s * ZPH, ZPH)], degv)
                pltpu.sync_copy(degv, degsh.at[pl.ds(s * ZPH, ZPH)])
            plsc.subcore_barrier()

            def iter_(t, carry):
                row0 = p * (NW * CPW) + w * CPW + t * NPI
                pltpu.sync_copy(srcall.at[pl.ds(row0, NPI)], sidxa)
                pltpu.sync_copy(dstall.at[pl.ds(row0, NPI)], didxa)
                for q in range(NPI // 2):
                    j0, j1 = 2 * q, 2 * q + 1
                    pltpu.async_copy(
                        xw.at[sidxa.at[j0, pl.ds(0, CHUNK)]], b0w, gs0)
                    pltpu.async_copy(
                        xw.at[sidxa.at[j1, pl.ds(0, CHUNK)]], b1w, gs1)
                    for k in range(128 // 16):
                        sl = pl.ds(k * 16, 16)
                        didx0[sl] = didxa[j0, sl]
                        didx1[sl] = didxa[j1, sl]
                    pltpu.make_async_copy(
                        xw.at[pl.ds(0, CHUNK)], b0w, gs0).wait()
                    pltpu.async_copy(b0, shared.at[didx0], ssem, add=True)
                    if with_deg:
                        pltpu.async_copy(ones, degsh.at[didx0], ssem, add=True)
                    pltpu.make_async_copy(
                        xw.at[pl.ds(0, CHUNK)], b1w, gs1).wait()
                    pltpu.async_copy(b1, shared.at[didx1], ssem, add=True)
                    if with_deg:
                        pltpu.async_copy(ones, degsh.at[didx1], ssem, add=True)
                    pltpu.make_async_copy(b0, shared.at[didx0], ssem).wait()
                    pltpu.make_async_copy(b1, shared.at[didx1], ssem).wait()
                    if with_deg:
                        pltpu.make_async_copy(ones, degsh.at[didx0], ssem).wait()
                        pltpu.make_async_copy(ones, degsh.at[didx1], ssem).wait()
                return carry

            lax.fori_loop(0, ROUNDS, iter_, 0)
            plsc.subcore_barrier()

            @pl.when(s < NS - 1)
            def _():
                pltpu.sync_copy(shared.at[pl.ds(s * OUT_FULL, OUT_FULL)],
                                aggout.at[pl.ds(out_base + s * OUT_FULL, OUT_FULL)])

            @pl.when(s == NS - 1)
            def _():
                pltpu.sync_copy(shared.at[pl.ds(OUT_TOFF, OUT_TAIL)],
                                aggout.at[pl.ds(out_base + OUT_TOFF, OUT_TAIL)])

            if with_deg:
                @pl.when(s < NS - 1)
                def _():
                    pltpu.sync_copy(degsh.at[pl.ds(s * OUT_FULL, OUT_FULL)], degv.at[pl.ds(0, OUT_FULL)])
                    pltpu.sync_copy(degv.at[pl.ds(0, OUT_FULL)],
                                    degout.at[pl.ds(out_base + s * OUT_FULL, OUT_FULL)])

                @pl.when(s == NS - 1)
                def _():
                    pltpu.sync_copy(degsh.at[pl.ds(OUT_TOFF, OUT_TAIL)], degv.at[pl.ds(0, OUT_TAIL)])
                    pltpu.sync_copy(degv.at[pl.ds(0, OUT_TAIL)],
                                    degout.at[pl.ds(out_base + OUT_TOFF, OUT_TAIL)])

            plsc.subcore_barrier()

    return pl.kernel(body, mesh=_mesh(), out_type=tuple(out_type) if with_deg else out_type[0],
                     scratch_types=scratch)


@functools.cache
def _agg_deg():
    return _make_agg(True)


@functools.cache
def _agg():
    return _make_agg(False)


def _score_body(xw, u0, i0, out, uv, iv, xu, xi, sv, sem):
    c = lax.axis_index("c")
    s = lax.axis_index("s")
    base = (s * NC + c) * PPW
    pltpu.sync_copy(u0.at[pl.ds(base, PPW)], uv)
    pltpu.sync_copy(i0.at[pl.ds(base, PPW)], iv)
    pltpu.async_copy(xw.at[uv], xu, sem).wait()
    pltpu.async_copy(xw.at[iv], xi, sem).wait()

    def pair(p, carry):
        acc = jnp.zeros((16,), _f32)
        for k in range(D // 16):
            sl = pl.ds(k * 16, 16)
            acc = acc + xu[p, sl] * xi[p, sl]
        sv[p] = acc
        return carry

    lax.fori_loop(0, PPW, pair, 0)
    pltpu.sync_copy(sv, out.at[pl.ds(base, PPW)])


@functools.cache
def _make_score():
    return pl.kernel(
        _score_body, mesh=_mesh(),
        out_type=jax.ShapeDtypeStruct((B, 16), _f32),
        scratch_types=[
            pltpu.VMEM((PPW,), jnp.int32),
            pltpu.VMEM((PPW,), jnp.int32),
            pltpu.VMEM((PPW, D), _f32),
            pltpu.VMEM((PPW, D), _f32),
            pltpu.VMEM((PPW, 16), _f32),
            pltpu.SemaphoreType.DMA,
        ])


def _finish_body(pp_ref, out_ref):
    s = jnp.sum(pp_ref[...], axis=1, keepdims=True)
    out_ref[...] = jnp.broadcast_to(1.0 / (1.0 + jnp.exp(-s)), (B, 128))


def _finish(pp):
    return pl.pallas_call(
        _finish_body,
        grid=(1,),
        in_specs=[pl.BlockSpec((B, 16), lambda b: (0, 0))],
        out_specs=pl.BlockSpec((B, 128), lambda b: (0, 0)),
        out_shape=jax.ShapeDtypeStruct((B, 128), _f32),
    )(pp)


def _enc_body(feat_ref, w_ref, b_ref, out_ref):
    out_ref[...] = (jnp.dot(feat_ref[...], w_ref[...], preferred_element_type=_f32)
                    + b_ref[0:1, :])


def _enc(item_feat, enc_W, encb_pk):
    return pl.pallas_call(
        _enc_body,
        grid=(NI // RBLK,),
        in_specs=[
            pl.BlockSpec((RBLK, D), lambda b: (b, 0)),
            pl.BlockSpec((D, D), lambda b: (0, 0)),
            pl.BlockSpec((8, D), lambda b: (0, 0)),
        ],
        out_specs=pl.BlockSpec((RBLK, D), lambda b: (b, 0)),
        out_shape=jax.ShapeDtypeStruct((NI, D), _f32),
    )(item_feat, enc_W, encb_pk)


def _dense_body(agg_ref, x_ref, deg_ref, wl_ref, wr_ref, pk_ref, y_ref, st_ref):
    b = pl.program_id(0)
    deg = deg_ref[0] + deg_ref[1]
    inv = 1.0 / jnp.maximum(deg, 1.0)
    a = (agg_ref[0] + agg_ref[1]) * inv
    y = (jnp.dot(a, wl_ref[...], preferred_element_type=_f32)
         + jnp.dot(x_ref[...], wr_ref[...], preferred_element_type=_f32)
         + pk_ref[0:1, :])
    y_ref[...] = y

    @pl.when(b == 0)
    def _():
        st_ref[...] = jnp.zeros_like(st_ref)

    st_ref[...] += jnp.concatenate(
        [jnp.sum(y, axis=0, keepdims=True),
         jnp.sum(y * y, axis=0, keepdims=True),
         jnp.zeros((6, D), _f32)], axis=0)


def _dense(aggp, x, degp, wl, wr, pk):
    return pl.pallas_call(
        _dense_body,
        grid=(NBLK,),
        in_specs=[
            pl.BlockSpec((NC, RBLK, D), lambda b: (0, b, 0)),
            pl.BlockSpec((RBLK, D), lambda b: (b, 0)),
            pl.BlockSpec((NC, RBLK, 1), lambda b: (0, b, 0)),
            pl.BlockSpec((D, D), lambda b: (0, 0)),
            pl.BlockSpec((D, D), lambda b: (0, 0)),
            pl.BlockSpec((8, D), lambda b: (0, 0)),
        ],
        out_specs=[
            pl.BlockSpec((RBLK, D), lambda b: (b, 0)),
            pl.BlockSpec((8, D), lambda b: (0, 0)),
        ],
        out_shape=[
            jax.ShapeDtypeStruct((N, D), _f32),
            jax.ShapeDtypeStruct((8, D), _f32),
        ],
    )(aggp, x, degp, wl, wr, pk)


def _make_norm(relu):
    def body(y_ref, st_ref, pk_ref, out_ref):
        mean = st_ref[0:1, :] * (1.0 / N)
        ey2 = st_ref[1:2, :] * (1.0 / N)
        var = ey2 - mean * mean
        rstd = lax.rsqrt(var + 1e-5)
        scale = pk_ref[1:2, :] * rstd
        shift = pk_ref[2:3, :] - mean * scale
        part = y_ref[...] * scale + shift
        if relu:
            part = jnp.maximum(part, 0.0)
        out_ref[...] = part

    def call(y, st, pk):
        return pl.pallas_call(
            body,
            grid=(NBLK,),
            in_specs=[
                pl.BlockSpec((RBLK, D), lambda b: (b, 0)),
                pl.BlockSpec((8, D), lambda b: (0, 0)),
                pl.BlockSpec((8, D), lambda b: (0, 0)),
            ],
            out_specs=pl.BlockSpec((RBLK, D), lambda b: (b, 0)),
            out_shape=jax.ShapeDtypeStruct((N, D), _f32),
        )(y, st, pk)

    return call


_norm_relu = _make_norm(True)
_norm_id = _make_norm(False)


def kernel(users, items, edge_user, edge_item, item_feat, user_emb,
           enc_W, enc_b, Wl, Wr, bl, gamma, beta):
    i32 = jnp.int32
    eu = edge_user.astype(i32)
    ei = edge_item.astype(i32)
    pad = EHP - E
    trash = NU + jnp.arange(pad, dtype=i32) % (NSPH - NU)
    zpad = jnp.arange(pad, dtype=i32) % N
    src64 = jnp.concatenate([eu, zpad, ei + NU, zpad]).reshape(-1, CHUNK)
    srcall = jnp.concatenate([src64, jnp.zeros_like(src64)], axis=1)
    dl = jnp.concatenate([ei, trash, eu, trash])
    dstall = jnp.stack([2 * dl, 2 * dl + 1], axis=1).reshape(-1, 128)

    z2 = jnp.zeros((NSP2, 128), _f32)
    z1 = jnp.zeros((NSP2,), _f32)

    zrow = jnp.zeros((1, D), _f32)
    pk = []
    for i in range(NLAYERS):
        pk.append(jnp.concatenate(
            [bl[i][None, :], gamma[i][None, :], beta[i][None, :],
             jnp.zeros((5, D), _f32)], axis=0))
    encb_pk = jnp.concatenate([enc_b[None, :]] + [zrow] * 7, axis=0)

    xi = _enc(item_feat, enc_W, encb_pk)
    x = jnp.concatenate([user_emb, xi], axis=0)

    degp = None
    for i in range(NLAYERS):
        if i == 0:
            aggflat, deg = _agg_deg()(x, srcall, dstall, z2, z1)
            degp = deg.reshape(NC, N, 2)[:, :, 0:1]
        else:
            aggflat = _agg()(x, srcall, dstall, z2, z1)
        aggp = aggflat.reshape(NC, N, D)
        y, st = _dense(aggp, x, degp, Wl[i], Wr[i], pk[i])
        x = (_norm_relu if i < NLAYERS - 1 else _norm_id)(y, st, pk[i])

    u0 = users.astype(i32)
    it0 = items.astype(i32) + NU
    pp = _make_score()(x, u0, it0)
    return _finish(pp)[:, 0]
